# baseline (speedup 1.0000x reference)
import functools

import jax
import jax.numpy as jnp
from jax import lax
from jax.experimental import pallas as pl
from jax.experimental.pallas import tpu as pltpu

K = 15
NUM_CLASSES = 10
THRESHOLD = 0.7
CHUNK = 100
NCHUNK = 2000
NEG = -3.0e38


def _body(x_ref, boxes_ref, out_ref, scratch_ref):
    x = x_ref[0]
    chunkmax = jnp.max(x, axis=1)

    iota_nc = lax.broadcasted_iota(jnp.int32, (NCHUNK,), 0)
    lane_i = lax.broadcasted_iota(jnp.int32, (1, CHUNK), 1)

    cand_rows = []
    cand_idx = []
    cm = chunkmax
    for _ in range(K):
        m = jnp.max(cm)
        c = jnp.min(jnp.where(cm == m, iota_nc, NCHUNK))
        cand_rows.append(x_ref[0, pl.ds(c, 1), :])
        cand_idx.append(c * CHUNK + lane_i)
        cm = jnp.where(iota_nc == c, NEG, cm)
    cand = jnp.concatenate(cand_rows, axis=0)
    cidx = jnp.concatenate(cand_idx, axis=0)

    sel_val = []
    sel_idx = []
    big = jnp.int32(NCHUNK * CHUNK)
    for _ in range(K):
        m = jnp.max(cand)
        fi = jnp.min(jnp.where(cand == m, cidx, big))
        sel_val.append(m)
        sel_idx.append(fi)
        cand = jnp.where(cidx == fi, NEG, cand)

    rows = []
    for i in range(K):
        fi = sel_idx[i]
        score = jnp.float32(1.0) / (jnp.float32(1.0) + jnp.exp(-sel_val[i]))
        use = jnp.where(score >= jnp.float32(THRESHOLD),
                        jnp.float32(1.0), jnp.float32(0.0))
        bi = fi // NUM_CLASSES
        brow = boxes_ref[0, pl.ds(bi, 1), :]
        r = bi // NUM_CLASSES
        coff = (bi % NUM_CLASSES) * NUM_CLASSES
        row = x_ref[0, pl.ds(r, 1), :]
        l100 = lax.broadcasted_iota(jnp.int32, (NUM_CLASSES, CHUNK), 1)
        j10 = lax.broadcasted_iota(jnp.int32, (NUM_CLASSES, CHUNK), 0)
        mask = l100 == (coff + j10)
        rowb = jnp.broadcast_to(row, (NUM_CLASSES, CHUNK))
        lrow = jnp.sum(jnp.where(mask, rowb, 0.0), axis=1).reshape(1, NUM_CLASSES)
        prow = jnp.float32(1.0) / (jnp.float32(1.0) + jnp.exp(-lrow))
        rows.append(jnp.concatenate([brow, prow], axis=1) * use)
    player = jnp.concatenate(rows, axis=0)
    scratch_ref[...] = player

    keys = jnp.max(player[:, 0:1], axis=1)
    iota_k = lax.broadcasted_iota(jnp.int32, (K,), 0)
    for i in range(K):
        m = jnp.max(keys)
        p = jnp.min(jnp.where(keys == m, iota_k, K))
        out_ref[0, pl.ds(i, 1), :] = scratch_ref[pl.ds(p, 1), :]
        keys = jnp.where(iota_k == p, NEG, keys)


@jax.jit
def kernel(logits, boxes):
    B, N, C = logits.shape
    flat3 = logits.reshape(B, NCHUNK, CHUNK)
    out = pl.pallas_call(
        _body,
        grid=(B,),
        in_specs=[
            pl.BlockSpec((1, NCHUNK, CHUNK), lambda b: (b, 0, 0)),
            pl.BlockSpec((1, N, 4), lambda b: (b, 0, 0)),
        ],
        out_specs=pl.BlockSpec((1, K, 14), lambda b: (b, 0, 0)),
        out_shape=jax.ShapeDtypeStruct((B, K, 14), jnp.float32),
        scratch_shapes=[pltpu.VMEM((K, 14), jnp.float32)],
    )(flat3, boxes)
    return out
